# Initial kernel scaffold; baseline (speedup 1.0000x reference)
import jax
import jax.numpy as jnp
from jax.experimental import pallas as pl


def kernel(input, index, src):
    raise NotImplementedError("write your pallas kernel here")


# speedup vs baseline: 41.8069x; 41.8069x over previous
import jax
import jax.numpy as jnp
from jax import lax
from jax.experimental import pallas as pl
from jax.experimental.pallas import tpu as pltpu
from jax.experimental.pallas import tpu_sc as plsc

N = 10000
E = 320000
D = 128

NUM_CORES = 2
NUM_SUBCORES = 16
NUM_WORKERS = NUM_CORES * NUM_SUBCORES
CPT = D // NUM_WORKERS
WC = 4000
CHUNKS = E // WC
VECS = WC // 16

BT = 2560
BN = N


def _t1_body(idx_ref, src_ref, idxT_ref, srcT_ref):
    idxT_ref[...] = idx_ref[...].T
    srcT_ref[...] = src_ref[...].T


def _transpose_in(index, src):
    return pl.pallas_call(
        _t1_body,
        grid=(E // BT,),
        in_specs=[
            pl.BlockSpec((BT, D), lambda i: (i, 0)),
            pl.BlockSpec((BT, D), lambda i: (i, 0)),
        ],
        out_specs=[
            pl.BlockSpec((D, BT), lambda i: (0, i)),
            pl.BlockSpec((D, BT), lambda i: (0, i)),
        ],
        out_shape=[
            jax.ShapeDtypeStruct((D, E), jnp.int32),
            jax.ShapeDtypeStruct((D, E), jnp.float32),
        ],
    )(index, src)


def _t2_body(outT_ref, input_ref, out_ref):
    out_ref[...] = input_ref[...] + outT_ref[...].T


def _merge_out(outT, input):
    return pl.pallas_call(
        _t2_body,
        grid=(1,),
        in_specs=[
            pl.BlockSpec((D, N), lambda i: (0, 0)),
            pl.BlockSpec((N, D), lambda i: (0, 0)),
        ],
        out_specs=pl.BlockSpec((N, D), lambda i: (0, 0)),
        out_shape=jax.ShapeDtypeStruct((N, D), jnp.float32),
    )(outT, input)


def _sc_body(idxT_hbm, srcT_hbm, outT_hbm, acc, ibuf, sbuf):
    wid = lax.axis_index("s") * NUM_CORES + lax.axis_index("c")
    row0 = wid * CPT

    zeros16 = jnp.zeros((16,), jnp.float32)

    def zero_body(k, _):
        acc[pl.ds(k * 16, 16)] = zeros16
        return 0

    lax.fori_loop(0, (CPT * N) // 16, zero_body, 0, unroll=8)

    for c in range(CPT):
        base = jnp.full((16,), c * N, jnp.int32)

        def chunk_body(ci, _):
            e0 = ci * WC
            pltpu.sync_copy(idxT_hbm.at[row0 + c, pl.ds(e0, WC)], ibuf)
            pltpu.sync_copy(srcT_hbm.at[row0 + c, pl.ds(e0, WC)], sbuf)

            def vec_body(k, _):
                iv = ibuf[pl.ds(k * 16, 16)]
                sv = sbuf[pl.ds(k * 16, 16)]
                plsc.addupdate_scatter(acc, [iv + base], sv)
                return 0

            lax.fori_loop(0, VECS, vec_body, 0, unroll=8)
            return 0

        lax.fori_loop(0, CHUNKS, chunk_body, 0)

    for c in range(CPT):
        pltpu.sync_copy(acc.at[pl.ds(c * N, N)], outT_hbm.at[row0 + c, :])


def _sc_scatter(idxT, srcT):
    mesh = plsc.VectorSubcoreMesh(core_axis_name="c", subcore_axis_name="s")
    f = pl.kernel(
        _sc_body,
        out_type=jax.ShapeDtypeStruct((D, N), jnp.float32),
        mesh=mesh,
        scratch_types=[
            pltpu.VMEM((CPT * N,), jnp.float32),
            pltpu.VMEM((WC,), jnp.int32),
            pltpu.VMEM((WC,), jnp.float32),
        ],
        compiler_params=pltpu.CompilerParams(
            use_tc_tiling_on_sc=False, needs_layout_passes=False
        ),
    )
    return f(idxT, srcT)


@jax.jit
def kernel(input, index, src):
    idxT, srcT = _transpose_in(index, src)
    outT = _sc_scatter(idxT, srcT)
    return _merge_out(outT, input)


# speedup vs baseline: 59.2436x; 1.4171x over previous
import jax
import jax.numpy as jnp
from jax import lax
from jax.experimental import pallas as pl
from jax.experimental.pallas import tpu as pltpu
from jax.experimental.pallas import tpu_sc as plsc

N = 10000
E = 320000
D = 128

NUM_CORES = 2
NUM_SUBCORES = 16
NUM_WORKERS = NUM_CORES * NUM_SUBCORES
CPT = D // NUM_WORKERS
WC = 8000
CHUNKS = E // WC
TOT = CPT * CHUNKS
VECS = WC // 16

BT = 2560
BN = N


def _t1_body(idx_ref, src_ref, idxT_ref, srcT_ref):
    idxT_ref[...] = idx_ref[...].T
    srcT_ref[...] = src_ref[...].T


def _transpose_in(index, src):
    return pl.pallas_call(
        _t1_body,
        grid=(E // BT,),
        in_specs=[
            pl.BlockSpec((BT, D), lambda i: (i, 0)),
            pl.BlockSpec((BT, D), lambda i: (i, 0)),
        ],
        out_specs=[
            pl.BlockSpec((D, BT), lambda i: (0, i)),
            pl.BlockSpec((D, BT), lambda i: (0, i)),
        ],
        out_shape=[
            jax.ShapeDtypeStruct((D, E), jnp.int32),
            jax.ShapeDtypeStruct((D, E), jnp.float32),
        ],
    )(index, src)


def _t2_body(outT_ref, input_ref, out_ref):
    out_ref[...] = input_ref[...] + outT_ref[...].T


def _merge_out(outT, input):
    return pl.pallas_call(
        _t2_body,
        grid=(1,),
        in_specs=[
            pl.BlockSpec((D, N), lambda i: (0, 0)),
            pl.BlockSpec((N, D), lambda i: (0, 0)),
        ],
        out_specs=pl.BlockSpec((N, D), lambda i: (0, 0)),
        out_shape=jax.ShapeDtypeStruct((N, D), jnp.float32),
    )(outT, input)


def _sc_body(idxT_hbm, srcT_hbm, outT_hbm, acc,
             ib0, ib1, sb0, sb1, si0, si1, ss0, ss1):
    wid = lax.axis_index("s") * NUM_CORES + lax.axis_index("c")
    row0 = wid * CPT

    zeros16 = jnp.zeros((16,), jnp.float32)

    def zero_body(k, _):
        acc[pl.ds(k * 16, 16)] = zeros16
        return 0

    lax.fori_loop(0, (CPT * N) // 16, zero_body, 0, unroll=8)

    def issue(s, ib, sb, isem, ssem):
        c = lax.div(s, CHUNKS)
        e0 = lax.rem(s, CHUNKS) * WC
        pltpu.async_copy(idxT_hbm.at[row0 + c, pl.ds(e0, WC)], ib, isem)
        pltpu.async_copy(srcT_hbm.at[row0 + c, pl.ds(e0, WC)], sb, ssem)

    def wait_pair(ib, sb, isem, ssem):
        pltpu.make_async_copy(idxT_hbm.at[row0, pl.ds(0, WC)], ib, isem).wait()
        pltpu.make_async_copy(srcT_hbm.at[row0, pl.ds(0, WC)], sb, ssem).wait()

    def compute(s, ib, sb):
        base = jnp.full((16,), lax.div(s, CHUNKS) * N, jnp.int32)

        def vec_body(k, _):
            iv = ib[pl.ds(k * 16, 16)]
            sv = sb[pl.ds(k * 16, 16)]
            plsc.addupdate_scatter(acc, [iv + base], sv)
            return 0

        lax.fori_loop(0, VECS, vec_body, 0, unroll=8)

    issue(jnp.int32(0), ib0, sb0, si0, ss0)
    issue(jnp.int32(1), ib1, sb1, si1, ss1)

    def step2(s2, _):
        s = s2 * 2
        wait_pair(ib0, sb0, si0, ss0)
        compute(s, ib0, sb0)

        @pl.when(s + 2 < TOT)
        def _():
            issue(s + 2, ib0, sb0, si0, ss0)

        wait_pair(ib1, sb1, si1, ss1)
        compute(s + 1, ib1, sb1)

        @pl.when(s + 3 < TOT)
        def _():
            issue(s + 3, ib1, sb1, si1, ss1)

        return 0

    lax.fori_loop(0, TOT // 2, step2, 0)

    for c in range(CPT):
        pltpu.sync_copy(acc.at[pl.ds(c * N, N)], outT_hbm.at[row0 + c, :])


def _sc_scatter(idxT, srcT):
    mesh = plsc.VectorSubcoreMesh(core_axis_name="c", subcore_axis_name="s")
    f = pl.kernel(
        _sc_body,
        out_type=jax.ShapeDtypeStruct((D, N), jnp.float32),
        mesh=mesh,
        scratch_types=[
            pltpu.VMEM((CPT * N,), jnp.float32),
            pltpu.VMEM((WC,), jnp.int32),
            pltpu.VMEM((WC,), jnp.int32),
            pltpu.VMEM((WC,), jnp.float32),
            pltpu.VMEM((WC,), jnp.float32),
            pltpu.SemaphoreType.DMA,
            pltpu.SemaphoreType.DMA,
            pltpu.SemaphoreType.DMA,
            pltpu.SemaphoreType.DMA,
        ],
        compiler_params=pltpu.CompilerParams(
            use_tc_tiling_on_sc=False, needs_layout_passes=False
        ),
    )
    return f(idxT, srcT)


@jax.jit
def kernel(input, index, src):
    idxT, srcT = _transpose_in(index, src)
    outT = _sc_scatter(idxT, srcT)
    return _merge_out(outT, input)


# speedup vs baseline: 88.3023x; 1.4905x over previous
import jax
import jax.numpy as jnp
from jax import lax
from jax.experimental import pallas as pl
from jax.experimental.pallas import tpu as pltpu
from jax.experimental.pallas import tpu_sc as plsc

N = 10000
E = 320000
D = 128

NUM_CORES = 2
NUM_SUBCORES = 16
NUM_WORKERS = NUM_CORES * NUM_SUBCORES
CPT = D // NUM_WORKERS
WC = 8000
CHUNKS = E // WC
TOT = CPT * CHUNKS
VECS = WC // 16

BT = 2560
BN = N


def _t1_body(idx_ref, src_ref, idxT_ref, srcT_ref):
    idxT_ref[...] = idx_ref[...].T
    srcT_ref[...] = src_ref[...].T


def _transpose_in(index, src):
    return pl.pallas_call(
        _t1_body,
        grid=(E // BT,),
        in_specs=[
            pl.BlockSpec((BT, D), lambda i: (i, 0)),
            pl.BlockSpec((BT, D), lambda i: (i, 0)),
        ],
        out_specs=[
            pl.BlockSpec((D, BT), lambda i: (0, i)),
            pl.BlockSpec((D, BT), lambda i: (0, i)),
        ],
        out_shape=[
            jax.ShapeDtypeStruct((D, E), jnp.int32),
            jax.ShapeDtypeStruct((D, E), jnp.float32),
        ],
    )(index, src)


def _t2_body(outT_ref, input_ref, out_ref):
    out_ref[...] = input_ref[...] + outT_ref[...].T


def _merge_out(outT, input):
    return pl.pallas_call(
        _t2_body,
        grid=(1,),
        in_specs=[
            pl.BlockSpec((D, N), lambda i: (0, 0)),
            pl.BlockSpec((N, D), lambda i: (0, 0)),
        ],
        out_specs=pl.BlockSpec((N, D), lambda i: (0, 0)),
        out_shape=jax.ShapeDtypeStruct((N, D), jnp.float32),
    )(outT, input)


def _sc_body(idxT_hbm, srcT_hbm, outT_hbm, acc,
             ib0, ib1, sb0, sb1, si0, si1, ss0, ss1):
    wid = lax.axis_index("s") * NUM_CORES + lax.axis_index("c")
    row0 = wid * CPT

    zeros16 = jnp.zeros((16,), jnp.float32)

    def zero_body(k, _):
        acc[pl.ds(k * 16, 16)] = zeros16
        return 0

    lax.fori_loop(0, (CPT * N) // 16, zero_body, 0, unroll=8)

    def issue(s, ib, sb, isem, ssem):
        c = lax.div(s, CHUNKS)
        e0 = lax.rem(s, CHUNKS) * WC
        pltpu.async_copy(idxT_hbm.at[row0 + c, pl.ds(e0, WC)], ib, isem)
        pltpu.async_copy(srcT_hbm.at[row0 + c, pl.ds(e0, WC)], sb, ssem)

    def wait_pair(ib, sb, isem, ssem):
        pltpu.make_async_copy(idxT_hbm.at[row0, pl.ds(0, WC)], ib, isem).wait()
        pltpu.make_async_copy(srcT_hbm.at[row0, pl.ds(0, WC)], sb, ssem).wait()

    def compute(s, ib, sb):
        base = jnp.full((16,), lax.div(s, CHUNKS) * N, jnp.int32)

        @plsc.parallel_loop(0, WC, step=16, unroll=8)
        def _(i):
            iv = ib[pl.ds(i, 16)]
            sv = sb[pl.ds(i, 16)]
            plsc.addupdate_scatter(acc, [iv + base], sv)

    issue(jnp.int32(0), ib0, sb0, si0, ss0)
    issue(jnp.int32(1), ib1, sb1, si1, ss1)

    def step2(s2, _):
        s = s2 * 2
        wait_pair(ib0, sb0, si0, ss0)
        compute(s, ib0, sb0)

        @pl.when(s + 2 < TOT)
        def _():
            issue(s + 2, ib0, sb0, si0, ss0)

        wait_pair(ib1, sb1, si1, ss1)
        compute(s + 1, ib1, sb1)

        @pl.when(s + 3 < TOT)
        def _():
            issue(s + 3, ib1, sb1, si1, ss1)

        return 0

    lax.fori_loop(0, TOT // 2, step2, 0)

    for c in range(CPT):
        pltpu.sync_copy(acc.at[pl.ds(c * N, N)], outT_hbm.at[row0 + c, :])


def _sc_scatter(idxT, srcT):
    mesh = plsc.VectorSubcoreMesh(core_axis_name="c", subcore_axis_name="s")
    f = pl.kernel(
        _sc_body,
        out_type=jax.ShapeDtypeStruct((D, N), jnp.float32),
        mesh=mesh,
        scratch_types=[
            pltpu.VMEM((CPT * N,), jnp.float32),
            pltpu.VMEM((WC,), jnp.int32),
            pltpu.VMEM((WC,), jnp.int32),
            pltpu.VMEM((WC,), jnp.float32),
            pltpu.VMEM((WC,), jnp.float32),
            pltpu.SemaphoreType.DMA,
            pltpu.SemaphoreType.DMA,
            pltpu.SemaphoreType.DMA,
            pltpu.SemaphoreType.DMA,
        ],
        compiler_params=pltpu.CompilerParams(
            use_tc_tiling_on_sc=False, needs_layout_passes=False
        ),
    )
    return f(idxT, srcT)


@jax.jit
def kernel(input, index, src):
    idxT, srcT = _transpose_in(index, src)
    outT = _sc_scatter(idxT, srcT)
    return _merge_out(outT, input)


# speedup vs baseline: 156.6776x; 1.7743x over previous
import jax
import jax.numpy as jnp
from jax import lax
from jax.experimental import pallas as pl
from jax.experimental.pallas import tpu as pltpu
from jax.experimental.pallas import tpu_sc as plsc

N = 10000
E = 320000
D = 128

NUM_CORES = 2
NUM_SUBCORES = 16
CPT = 8
NGROUP = D // CPT
EH = E // 2
WR = 1000
CHUNKS = EH // WR
VECS = (WR * CPT) // 16
MB = 8000


def _t2_body(outT0_ref, outT1_ref, input_ref, out_ref):
    out_ref[...] = (input_ref[...] + outT0_ref[...].T) + outT1_ref[...].T


def _merge_out(outT0, outT1, input):
    return pl.pallas_call(
        _t2_body,
        grid=(1,),
        in_specs=[
            pl.BlockSpec((D, N), lambda i: (0, 0)),
            pl.BlockSpec((D, N), lambda i: (0, 0)),
            pl.BlockSpec((N, D), lambda i: (0, 0)),
        ],
        out_specs=pl.BlockSpec((N, D), lambda i: (0, 0)),
        out_shape=jax.ShapeDtypeStruct((N, D), jnp.float32),
    )(outT0, outT1, input)


def _sc_body(idx_hbm, src_hbm, outT0_hbm, outT1_hbm, acc,
             ib0, ib1, sb0, sb1, si0, si1, ss0, ss1):
    core = lax.axis_index("c")
    sub = lax.axis_index("s")
    grp = sub % 8
    half = sub // 8
    gg = core * 8 + grp
    col0 = gg * CPT
    r0 = half * EH

    zeros16 = jnp.zeros((16,), jnp.float32)

    def zero_body(k, _):
        acc[pl.ds(k * 16, 16)] = zeros16
        return 0

    lax.fori_loop(0, (CPT * N) // 16, zero_body, 0, unroll=8)

    lane = lax.iota(jnp.int32, 16)
    cvec = lane % 8
    rpat = lane // 8
    cbase = cvec * N

    def issue(s, ib, sb, isem, ssem):
        row = r0 + s * WR
        pltpu.async_copy(idx_hbm.at[pl.ds(row, WR), pl.ds(col0, CPT)], ib, isem)
        pltpu.async_copy(src_hbm.at[pl.ds(row, WR), pl.ds(col0, CPT)], sb, ssem)

    def wait_pair(ib, sb, isem, ssem):
        pltpu.make_async_copy(
            idx_hbm.at[pl.ds(0, WR), pl.ds(col0, CPT)], ib, isem).wait()
        pltpu.make_async_copy(
            src_hbm.at[pl.ds(0, WR), pl.ds(col0, CPT)], sb, ssem).wait()

    def compute(ib, sb):
        @plsc.parallel_loop(0, VECS, step=1, unroll=8)
        def _(t):
            rvec = rpat + jnp.full((16,), t * 2, jnp.int32)
            iv = plsc.load_gather(ib, [rvec, cvec])
            sv = plsc.load_gather(sb, [rvec, cvec])
            plsc.addupdate_scatter(acc, [iv + cbase], sv)

    issue(jnp.int32(0), ib0, sb0, si0, ss0)
    issue(jnp.int32(1), ib1, sb1, si1, ss1)

    def step2(s2, _):
        s = s2 * 2
        wait_pair(ib0, sb0, si0, ss0)
        compute(ib0, sb0)

        @pl.when(s + 2 < CHUNKS)
        def _():
            issue(s + 2, ib0, sb0, si0, ss0)

        wait_pair(ib1, sb1, si1, ss1)
        compute(ib1, sb1)

        @pl.when(s + 3 < CHUNKS)
        def _():
            issue(s + 3, ib1, sb1, si1, ss1)

        return 0

    lax.fori_loop(0, CHUNKS // 2, step2, 0)

    @pl.when(half == 0)
    def _():
        for c in range(CPT):
            pltpu.sync_copy(acc.at[pl.ds(c * N, N)], outT0_hbm.at[col0 + c, :])

    @pl.when(half == 1)
    def _():
        for c in range(CPT):
            pltpu.sync_copy(acc.at[pl.ds(c * N, N)], outT1_hbm.at[col0 + c, :])


def _sc_scatter(index, src):
    mesh = plsc.VectorSubcoreMesh(core_axis_name="c", subcore_axis_name="s")
    f = pl.kernel(
        _sc_body,
        out_type=[
            jax.ShapeDtypeStruct((D, N), jnp.float32),
            jax.ShapeDtypeStruct((D, N), jnp.float32),
        ],
        mesh=mesh,
        scratch_types=[
            pltpu.VMEM((CPT * N,), jnp.float32),
            pltpu.VMEM((WR, CPT), jnp.int32),
            pltpu.VMEM((WR, CPT), jnp.int32),
            pltpu.VMEM((WR, CPT), jnp.float32),
            pltpu.VMEM((WR, CPT), jnp.float32),
            pltpu.SemaphoreType.DMA,
            pltpu.SemaphoreType.DMA,
            pltpu.SemaphoreType.DMA,
            pltpu.SemaphoreType.DMA,
        ],
        compiler_params=pltpu.CompilerParams(
            use_tc_tiling_on_sc=False, needs_layout_passes=False
        ),
    )
    return f(index, src)


@jax.jit
def kernel(input, index, src):
    outT0, outT1 = _sc_scatter(index, src)
    return _merge_out(outT0, outT1, input)


# speedup vs baseline: 158.1704x; 1.0095x over previous
import jax
import jax.numpy as jnp
from jax import lax
from jax.experimental import pallas as pl
from jax.experimental.pallas import tpu as pltpu
from jax.experimental.pallas import tpu_sc as plsc

N = 10000
E = 320000
D = 128

NUM_CORES = 2
NUM_SUBCORES = 16
CPT = 8
NGROUP = D // CPT
EH = E // 2
WR = 500
CHUNKS = EH // WR
VECS = (WR * CPT) // 16
NBUF = 4


def _t2_body(outT0_ref, outT1_ref, input_ref, out_ref):
    out_ref[...] = (input_ref[...] + outT0_ref[...].T) + outT1_ref[...].T


def _merge_out(outT0, outT1, input):
    return pl.pallas_call(
        _t2_body,
        grid=(1,),
        in_specs=[
            pl.BlockSpec((D, N), lambda i: (0, 0)),
            pl.BlockSpec((D, N), lambda i: (0, 0)),
            pl.BlockSpec((N, D), lambda i: (0, 0)),
        ],
        out_specs=pl.BlockSpec((N, D), lambda i: (0, 0)),
        out_shape=jax.ShapeDtypeStruct((N, D), jnp.float32),
    )(outT0, outT1, input)


def _sc_body(idx_hbm, src_hbm, outT0_hbm, outT1_hbm, acc,
             ib0, ib1, ib2, ib3, sb0, sb1, sb2, sb3,
             si0, si1, si2, si3, ss0, ss1, ss2, ss3):
    core = lax.axis_index("c")
    sub = lax.axis_index("s")
    grp = sub % 8
    half = sub // 8
    gg = core * 8 + grp
    col0 = gg * CPT
    r0 = half * EH

    zeros16 = jnp.zeros((16,), jnp.float32)

    def zero_body(k, _):
        acc[pl.ds(k * 16, 16)] = zeros16
        return 0

    lax.fori_loop(0, (CPT * N) // 16, zero_body, 0, unroll=8)

    lane = lax.iota(jnp.int32, 16)
    cvec = lane % 8
    rpat = lane // 8
    cbase = cvec * N

    def issue(s, ib, sb, isem, ssem):
        row = r0 + s * WR
        pltpu.async_copy(idx_hbm.at[pl.ds(row, WR), pl.ds(col0, CPT)], ib, isem)
        pltpu.async_copy(src_hbm.at[pl.ds(row, WR), pl.ds(col0, CPT)], sb, ssem)

    def wait_pair(ib, sb, isem, ssem):
        pltpu.make_async_copy(
            idx_hbm.at[pl.ds(0, WR), pl.ds(col0, CPT)], ib, isem).wait()
        pltpu.make_async_copy(
            src_hbm.at[pl.ds(0, WR), pl.ds(col0, CPT)], sb, ssem).wait()

    def compute(ib, sb):
        @plsc.parallel_loop(0, VECS, step=1, unroll=8)
        def _(t):
            rvec = rpat + jnp.full((16,), t * 2, jnp.int32)
            iv = plsc.load_gather(ib, [rvec, cvec])
            sv = plsc.load_gather(sb, [rvec, cvec])
            plsc.addupdate_scatter(acc, [iv + cbase], sv)

    bufs = [(ib0, sb0, si0, ss0), (ib1, sb1, si1, ss1),
            (ib2, sb2, si2, ss2), (ib3, sb3, si3, ss3)]

    for b in range(NBUF):
        issue(jnp.int32(b), *bufs[b])

    def stepn(sn, _):
        s = sn * NBUF
        for b in range(NBUF):
            ib, sb, isem, ssem = bufs[b]
            wait_pair(ib, sb, isem, ssem)
            compute(ib, sb)

            @pl.when(s + b + NBUF < CHUNKS)
            def _():
                issue(s + b + NBUF, ib, sb, isem, ssem)

        return 0

    lax.fori_loop(0, CHUNKS // NBUF, stepn, 0)

    @pl.when(half == 0)
    def _():
        for c in range(CPT):
            pltpu.sync_copy(acc.at[pl.ds(c * N, N)], outT0_hbm.at[col0 + c, :])

    @pl.when(half == 1)
    def _():
        for c in range(CPT):
            pltpu.sync_copy(acc.at[pl.ds(c * N, N)], outT1_hbm.at[col0 + c, :])


def _sc_scatter(index, src):
    mesh = plsc.VectorSubcoreMesh(core_axis_name="c", subcore_axis_name="s")
    f = pl.kernel(
        _sc_body,
        out_type=[
            jax.ShapeDtypeStruct((D, N), jnp.float32),
            jax.ShapeDtypeStruct((D, N), jnp.float32),
        ],
        mesh=mesh,
        scratch_types=[
            pltpu.VMEM((CPT * N,), jnp.float32),
            pltpu.VMEM((WR, CPT), jnp.int32),
            pltpu.VMEM((WR, CPT), jnp.int32),
            pltpu.VMEM((WR, CPT), jnp.int32),
            pltpu.VMEM((WR, CPT), jnp.int32),
            pltpu.VMEM((WR, CPT), jnp.float32),
            pltpu.VMEM((WR, CPT), jnp.float32),
            pltpu.VMEM((WR, CPT), jnp.float32),
            pltpu.VMEM((WR, CPT), jnp.float32),
            pltpu.SemaphoreType.DMA,
            pltpu.SemaphoreType.DMA,
            pltpu.SemaphoreType.DMA,
            pltpu.SemaphoreType.DMA,
            pltpu.SemaphoreType.DMA,
            pltpu.SemaphoreType.DMA,
            pltpu.SemaphoreType.DMA,
            pltpu.SemaphoreType.DMA,
        ],
        compiler_params=pltpu.CompilerParams(
            use_tc_tiling_on_sc=False, needs_layout_passes=False
        ),
    )
    return f(index, src)


@jax.jit
def kernel(input, index, src):
    outT0, outT1 = _sc_scatter(index, src)
    return _merge_out(outT0, outT1, input)
